# Initial kernel scaffold; baseline (speedup 1.0000x reference)
import jax
import jax.numpy as jnp
from jax.experimental import pallas as pl


def kernel(dynamic, static, chain_delays, turnaround_times, params, database):
    raise NotImplementedError("write your pallas kernel here")


# speedup vs baseline: 1.7845x; 1.7845x over previous
import functools

import jax
import jax.numpy as jnp
from jax import lax
from jax.experimental import pallas as pl
from jax.experimental.pallas import tpu as pltpu
from jax.experimental.pallas import tpu_sc as plsc

TOP_K = 5
ALPHA = 0.5
_PREC = None


def _mm(a, b):
    return lax.dot_general(a, b, (((a.ndim - 1,), (0,)), ((), ())),
                           precision=_PREC, preferred_element_type=jnp.float32)


def _shift_time(x, s):
    if s == 0:
        return x
    bb, ll, cc = x.shape
    z = jnp.zeros((bb, s, cc), x.dtype)
    return jnp.concatenate([z, x[:, :ll - s, :]], axis=1)


def _grn_block(x, w1, b1, w2, b2, wg, bg, wv, bv, ln_g, ln_b, wskip):
    skip = _mm(x, wskip) if wskip is not None else x
    h0 = _mm(x, w1) + b1
    h = jnp.where(h0 > 0, h0, jnp.exp(jnp.minimum(h0, 0.0)) - 1.0)
    h = _mm(h, w2) + b2
    glu = jax.nn.sigmoid(_mm(h, wg) + bg) * (_mm(h, wv) + bv)
    y = skip + glu
    mu = jnp.mean(y, axis=-1, keepdims=True)
    var = jnp.var(y, axis=-1, keepdims=True)
    return ln_g * (y - mu) / jnp.sqrt(var + 1e-5) + ln_b


def _prefix_body(dyn_ref, static_ref, cd_ref, ta_ref, beta_ref,
                 wc0_ref, b0_ref, wr0_ref, wc1_ref, b1_ref, wr1_ref,
                 wc2_ref, b2_ref,
                 gs_refs, gc_refs,
                 hcur_out, hglob_out, qn_out, yprop_out, beta_out):
    x = dyn_ref[...]
    dil = (1, 2, 4)
    convs = ((wc0_ref, b0_ref, wr0_ref), (wc1_ref, b1_ref, wr1_ref),
             (wc2_ref, b2_ref, None))
    for i, (wc, b, wr) in enumerate(convs):
        d = dil[i]
        xcat = jnp.concatenate(
            [_shift_time(x, 2 * d), _shift_time(x, d), x], axis=2)
        bb, ll, c3 = xcat.shape
        y = jax.nn.relu(_mm(xcat.reshape(bb * ll, c3), wc[...]) + b[...])
        x2d = x.reshape(bb * ll, x.shape[2])
        res = _mm(x2d, wr[...]) if wr is not None else x2d
        x = (y + res).reshape(bb, ll, y.shape[-1])

    h_cur = x[:, x.shape[1] - 1, :]
    h_glob = jnp.mean(x, axis=1)

    gs = [r[...] for r in gs_refs]
    h_static = _grn_block(static_ref[...], *gs[:10], gs[10])
    gc = [r[...] for r in gc_refs]
    h_cur = h_cur + _grn_block(h_static, *gc[:10], None)
    h_glob = h_glob + h_static

    nrm = jnp.sqrt(jnp.sum(h_cur * h_cur, axis=-1, keepdims=True))
    qn_out[...] = h_cur / (nrm + 1e-8)
    hcur_out[...] = h_cur
    hglob_out[...] = h_glob

    beta_pos = jax.nn.softplus(beta_ref[...])
    beta_out[...] = beta_pos
    decay = jnp.exp(-beta_pos[0, 0] * ta_ref[...])
    yprop_out[...] = jnp.sum(cd_ref[...] * decay, axis=1, keepdims=True)


def _run_prefix(dynamic, static, chain_delays, turnaround_times, params):
    b, l, nd = dynamic.shape
    d = 128
    bb = 128 if b % 128 == 0 else b
    grid = (b // bb,)

    tcn = params['tcn']
    wcs, bs, wrs = [], [], []
    for lyr in tcn:
        w = lyr['W']
        wcs.append(jnp.transpose(w, (2, 1, 0)).reshape(-1, w.shape[0]))
        bs.append(lyr['b'].reshape(1, -1))
        wrs.append(jnp.transpose(lyr['Wres']) if 'Wres' in lyr else None)

    def grn_flat(p, skip):
        out = [p['W1'], p['b1'].reshape(1, -1), p['W2'], p['b2'].reshape(1, -1),
               p['Wg'], p['bg'].reshape(1, -1), p['Wv'], p['bv'].reshape(1, -1),
               p['ln_g'].reshape(1, -1), p['ln_b'].reshape(1, -1)]
        if skip:
            out.append(p['Wskip'])
        return out

    gs_list = grn_flat(params['grn_static'], True)
    gc_list = grn_flat(params['static_ctx'], False)

    beta = params['delay']['beta'].reshape(1, 1)

    operands = ([dynamic, static, chain_delays, turnaround_times, beta,
                 wcs[0], bs[0], wrs[0], wcs[1], bs[1], wrs[1], wcs[2], bs[2]]
                + gs_list + gc_list)

    def bspec(arr, mapped):
        nd_ = arr.ndim
        if mapped:
            shape = (bb,) + arr.shape[1:]
            return pl.BlockSpec(shape, lambda i: (i,) + (0,) * (nd_ - 1))
        return pl.BlockSpec(arr.shape, lambda i: (0,) * nd_)

    in_specs = [bspec(dynamic, True), bspec(static, True),
                bspec(chain_delays, True), bspec(turnaround_times, True),
                bspec(beta, False)]
    in_specs += [bspec(a, False) for a in operands[5:]]

    out_shape = [jax.ShapeDtypeStruct((b, d), jnp.float32),
                 jax.ShapeDtypeStruct((b, d), jnp.float32),
                 jax.ShapeDtypeStruct((b, d), jnp.float32),
                 jax.ShapeDtypeStruct((b, 1), jnp.float32),
                 jax.ShapeDtypeStruct((1, 1), jnp.float32)]
    out_specs = [pl.BlockSpec((bb, d), lambda i: (i, 0)),
                 pl.BlockSpec((bb, d), lambda i: (i, 0)),
                 pl.BlockSpec((bb, d), lambda i: (i, 0)),
                 pl.BlockSpec((bb, 1), lambda i: (i, 0)),
                 pl.BlockSpec((1, 1), lambda i: (0, 0))]

    def body(*refs):
        (dyn, st, cd, ta, bt, wc0, b0, wr0, wc1, b1, wr1, wc2, b2) = refs[:13]
        gs_refs = refs[13:24]
        gc_refs = refs[24:34]
        outs = refs[34:]
        _prefix_body(dyn, st, cd, ta, bt, wc0, b0, wr0, wc1, b1, wr1,
                     wc2, b2, gs_refs, gc_refs, *outs)

    return pl.pallas_call(
        body, grid=grid, in_specs=in_specs, out_specs=out_specs,
        out_shape=out_shape)(*operands)


def _topk_body(qn_ref, db_ref, vals_out, idx_out, vals_s, idx_s, *, nsteps, nb):
    j = pl.program_id(0)

    @pl.when(j == 0)
    def _init():
        vals_s[...] = jnp.full(vals_s.shape, -jnp.inf, jnp.float32)
        idx_s[...] = jnp.zeros(idx_s.shape, jnp.int32)

    db = db_ref[...]
    nrm = jnp.sqrt(jnp.sum(db * db, axis=-1, keepdims=True))
    dbn = db / (nrm + 1e-8)
    s = lax.dot_general(qn_ref[...], dbn, (((1,), (1,)), ((), ())),
                        precision=_PREC, preferred_element_type=jnp.float32)

    bq = s.shape[0]
    col = lax.broadcasted_iota(jnp.int32, (bq, nb), 1)
    bvs, bis = [], []
    for _ in range(TOP_K):
        m = jnp.max(s, axis=1, keepdims=True)
        am = jnp.min(jnp.where(s == m, col, nb), axis=1, keepdims=True)
        bvs.append(m)
        bis.append(am + j * nb)
        s = jnp.where(col == am, -jnp.inf, s)

    cand_v = jnp.concatenate([vals_s[...]] + bvs, axis=1)
    cand_i = jnp.concatenate([idx_s[...]] + bis, axis=1)
    col10 = lax.broadcasted_iota(jnp.int32, cand_v.shape, 1)
    nvs, nis = [], []
    for _ in range(TOP_K):
        m = jnp.max(cand_v, axis=1, keepdims=True)
        am = jnp.min(jnp.where(cand_v == m, col10, 2 * TOP_K),
                     axis=1, keepdims=True)
        sel = col10 == am
        ci = jnp.sum(jnp.where(sel, cand_i, 0), axis=1, keepdims=True)
        nvs.append(m)
        nis.append(ci)
        cand_v = jnp.where(sel, -jnp.inf, cand_v)
    vals_s[...] = jnp.concatenate(nvs, axis=1)
    idx_s[...] = jnp.concatenate(nis, axis=1)

    @pl.when(j == nsteps - 1)
    def _fin():
        vals_out[...] = vals_s[...]
        idx_out[...] = idx_s[...]


def _run_topk(qn, database):
    b, d = qn.shape
    db_rows = database.shape[0]
    nb = 2000 if db_rows % 2000 == 0 else db_rows
    nsteps = db_rows // nb

    return pl.pallas_call(
        functools.partial(_topk_body, nsteps=nsteps, nb=nb),
        grid=(nsteps,),
        in_specs=[pl.BlockSpec((b, d), lambda j: (0, 0)),
                  pl.BlockSpec((nb, d), lambda j: (j, 0))],
        out_specs=[pl.BlockSpec((b, TOP_K), lambda j: (0, 0)),
                   pl.BlockSpec((b, TOP_K), lambda j: (0, 0))],
        out_shape=[jax.ShapeDtypeStruct((b, TOP_K), jnp.float32),
                   jax.ShapeDtypeStruct((b, TOP_K), jnp.int32)],
        scratch_shapes=[pltpu.VMEM((b, TOP_K), jnp.float32),
                        pltpu.VMEM((b, TOP_K), jnp.int32)],
    )(qn, database)


def _run_gather_sc(database, flat_idx):
    n = flat_idx.shape[0]
    d = database.shape[1]
    info = plsc.get_sparse_core_info()
    nw = info.num_cores * info.num_subcores
    b_per_w = n // nw
    nchunks = 1
    while b_per_w // nchunks > 128 or (b_per_w // nchunks) % 8 != 0:
        nchunks += 1
    chunk = b_per_w // nchunks
    num_cores = info.num_cores

    @functools.partial(
        pl.kernel,
        mesh=plsc.VectorSubcoreMesh(core_axis_name="c", subcore_axis_name="s"),
        out_type=jax.ShapeDtypeStruct((n, d), jnp.float32),
        scratch_types=[pltpu.VMEM((b_per_w,), jnp.int32),
                       pltpu.VMEM((b_per_w, d), jnp.float32),
                       pltpu.SemaphoreType.DMA],
    )
    def gather(table_hbm, idx_hbm, out_hbm, idx_v, rows_v, sem):
        wid = lax.axis_index("s") * num_cores + lax.axis_index("c")
        base = wid * b_per_w
        pltpu.sync_copy(idx_hbm.at[pl.ds(base, b_per_w)], idx_v)
        for c in range(nchunks):
            pltpu.async_copy(table_hbm.at[idx_v.at[pl.ds(c * chunk, chunk)]],
                             rows_v.at[pl.ds(c * chunk, chunk)], sem).wait()
        pltpu.sync_copy(rows_v, out_hbm.at[pl.ds(base, b_per_w)])

    return gather(database, flat_idx)


def _tail_body(hcur_ref, hglob_ref, vals_ref, rows_ref, yprop_ref,
               fw1, fb1, fw2, fb2, fwo, fbo, dwp, dbp,
               hw1, hb1, hw2, hb2, hw3, hb3,
               pred_out, hf_out):
    v = vals_ref[...]
    w = jax.nn.softmax(v, axis=-1)
    rows = rows_ref[...]
    h_ret = jnp.zeros((rows.shape[0], rows.shape[2]), jnp.float32)
    for i in range(TOP_K):
        h_ret = h_ret + w[:, i:i + 1] * rows[:, i, :]

    h_cur = hcur_ref[...]
    h_glob = hglob_ref[...]
    h_f = ALPHA * h_ret + (1.0 - ALPHA) * h_cur
    hf_out[...] = h_f

    ssum = h_cur + h_f + h_glob
    pooled = ssum / 3.0
    a = jax.nn.relu(_mm(pooled, fw1[...]) + fb1[...])
    a = jax.nn.sigmoid(_mm(a, fw2[...]) + fb2[...])
    h_fused = _mm(ssum * a, fwo[...]) + fbo[...]

    h_prop = jax.nn.relu(_mm(yprop_ref[...], dwp[...]) + dbp[...])
    h_final = jnp.concatenate([h_fused, h_prop], axis=-1)
    x = jax.nn.relu(_mm(h_final, hw1[...]) + hb1[...])
    x = jax.nn.relu(_mm(x, hw2[...]) + hb2[...])
    pred_out[...] = _mm(x, hw3[...]) + hb3[...]


def _run_tail(h_cur, h_glob, top_vals, rows, y_prop, params):
    b, d = h_cur.shape
    fp, dp, hp = params['fusion'], params['delay'], params['head']
    ops = [h_cur, h_glob, top_vals, rows, y_prop,
           fp['W1'], fp['b1'].reshape(1, -1), fp['W2'], fp['b2'].reshape(1, -1),
           fp['Wo'], fp['bo'].reshape(1, -1),
           dp['Wp'], dp['bp'].reshape(1, -1),
           hp['W1'], hp['b1'].reshape(1, -1), hp['W2'], hp['b2'].reshape(1, -1),
           hp['W3'], hp['b3'].reshape(1, -1)]
    in_specs = [pl.BlockSpec(a.shape, lambda _n=a.ndim: (0,) * _n) for a in ops]
    return pl.pallas_call(
        _tail_body,
        in_specs=in_specs,
        out_specs=[pl.BlockSpec((b, 1), lambda: (0, 0)),
                   pl.BlockSpec((b, d), lambda: (0, 0))],
        out_shape=[jax.ShapeDtypeStruct((b, 1), jnp.float32),
                   jax.ShapeDtypeStruct((b, d), jnp.float32)],
    )(*ops)


def kernel(dynamic, static, chain_delays, turnaround_times, params, database):
    h_cur, h_glob, qn, y_prop, beta_pos = _run_prefix(
        dynamic, static, chain_delays, turnaround_times, params)
    top_vals, top_idx = _run_topk(qn, database)
    rows = _run_gather_sc(database, top_idx.reshape(-1))
    rows = rows.reshape(top_idx.shape[0], TOP_K, database.shape[1])
    pred, h_f = _run_tail(h_cur, h_glob, top_vals, rows, y_prop, params)
    return (pred[:, 0], h_cur, h_glob, h_f, y_prop[:, 0],
            beta_pos.reshape(()))


# speedup vs baseline: 2.3783x; 1.3328x over previous
import functools

import jax
import jax.numpy as jnp
from jax import lax
from jax.experimental import pallas as pl
from jax.experimental.pallas import tpu as pltpu
from jax.experimental.pallas import tpu_sc as plsc

TOP_K = 5
ALPHA = 0.5
_PREC = None


def _mm(a, b):
    return lax.dot_general(a, b, (((a.ndim - 1,), (0,)), ((), ())),
                           precision=_PREC, preferred_element_type=jnp.float32)


def _shift_time(x, s):
    if s == 0:
        return x
    bb, ll, cc = x.shape
    z = jnp.zeros((bb, s, cc), x.dtype)
    return jnp.concatenate([z, x[:, :ll - s, :]], axis=1)


def _grn_block(x, w1, b1, w2, b2, wg, bg, wv, bv, ln_g, ln_b, wskip):
    skip = _mm(x, wskip) if wskip is not None else x
    h0 = _mm(x, w1) + b1
    h = jnp.where(h0 > 0, h0, jnp.exp(jnp.minimum(h0, 0.0)) - 1.0)
    h = _mm(h, w2) + b2
    glu = jax.nn.sigmoid(_mm(h, wg) + bg) * (_mm(h, wv) + bv)
    y = skip + glu
    mu = jnp.mean(y, axis=-1, keepdims=True)
    var = jnp.var(y, axis=-1, keepdims=True)
    return ln_g * (y - mu) / jnp.sqrt(var + 1e-5) + ln_b


def _prefix_body(dyn_ref, static_ref, cd_ref, ta_ref, beta_ref,
                 wc0_ref, b0_ref, wr0_ref, wc1_ref, b1_ref, wr1_ref,
                 wc2_ref, b2_ref,
                 gs_refs, gc_refs,
                 hcur_out, hglob_out, qn_out, yprop_out, beta_out):
    x = dyn_ref[...]
    dil = (1, 2, 4)
    convs = ((wc0_ref, b0_ref, wr0_ref), (wc1_ref, b1_ref, wr1_ref),
             (wc2_ref, b2_ref, None))
    for i, (wc, b, wr) in enumerate(convs):
        d = dil[i]
        xcat = jnp.concatenate(
            [_shift_time(x, 2 * d), _shift_time(x, d), x], axis=2)
        bb, ll, c3 = xcat.shape
        y = jax.nn.relu(_mm(xcat.reshape(bb * ll, c3), wc[...]) + b[...])
        x2d = x.reshape(bb * ll, x.shape[2])
        res = _mm(x2d, wr[...]) if wr is not None else x2d
        x = (y + res).reshape(bb, ll, y.shape[-1])

    h_cur = x[:, x.shape[1] - 1, :]
    h_glob = jnp.mean(x, axis=1)

    gs = [r[...] for r in gs_refs]
    h_static = _grn_block(static_ref[...], *gs[:10], gs[10])
    gc = [r[...] for r in gc_refs]
    h_cur = h_cur + _grn_block(h_static, *gc[:10], None)
    h_glob = h_glob + h_static

    nrm = jnp.sqrt(jnp.sum(h_cur * h_cur, axis=-1, keepdims=True))
    qn_out[...] = h_cur / (nrm + 1e-8)
    hcur_out[...] = h_cur
    hglob_out[...] = h_glob

    beta_pos = jax.nn.softplus(beta_ref[...])
    beta_out[...] = beta_pos
    decay = jnp.exp(-beta_pos[0, 0] * ta_ref[...])
    yprop_out[...] = jnp.sum(cd_ref[...] * decay, axis=1, keepdims=True)


def _run_prefix(dynamic, static, chain_delays, turnaround_times, params):
    b, l, nd = dynamic.shape
    d = 128
    bb = 128 if b % 128 == 0 else b
    grid = (b // bb,)

    tcn = params['tcn']
    wcs, bs, wrs = [], [], []
    for lyr in tcn:
        w = lyr['W']
        wcs.append(jnp.transpose(w, (2, 1, 0)).reshape(-1, w.shape[0]))
        bs.append(lyr['b'].reshape(1, -1))
        wrs.append(jnp.transpose(lyr['Wres']) if 'Wres' in lyr else None)

    def grn_flat(p, skip):
        out = [p['W1'], p['b1'].reshape(1, -1), p['W2'], p['b2'].reshape(1, -1),
               p['Wg'], p['bg'].reshape(1, -1), p['Wv'], p['bv'].reshape(1, -1),
               p['ln_g'].reshape(1, -1), p['ln_b'].reshape(1, -1)]
        if skip:
            out.append(p['Wskip'])
        return out

    gs_list = grn_flat(params['grn_static'], True)
    gc_list = grn_flat(params['static_ctx'], False)

    beta = params['delay']['beta'].reshape(1, 1)

    operands = ([dynamic, static, chain_delays, turnaround_times, beta,
                 wcs[0], bs[0], wrs[0], wcs[1], bs[1], wrs[1], wcs[2], bs[2]]
                + gs_list + gc_list)

    def bspec(arr, mapped):
        nd_ = arr.ndim
        if mapped:
            shape = (bb,) + arr.shape[1:]
            return pl.BlockSpec(shape, lambda i: (i,) + (0,) * (nd_ - 1))
        return pl.BlockSpec(arr.shape, lambda i: (0,) * nd_)

    in_specs = [bspec(dynamic, True), bspec(static, True),
                bspec(chain_delays, True), bspec(turnaround_times, True),
                bspec(beta, False)]
    in_specs += [bspec(a, False) for a in operands[5:]]

    out_shape = [jax.ShapeDtypeStruct((b, d), jnp.float32),
                 jax.ShapeDtypeStruct((b, d), jnp.float32),
                 jax.ShapeDtypeStruct((b, d), jnp.float32),
                 jax.ShapeDtypeStruct((b, 1), jnp.float32),
                 jax.ShapeDtypeStruct((1, 1), jnp.float32)]
    out_specs = [pl.BlockSpec((bb, d), lambda i: (i, 0)),
                 pl.BlockSpec((bb, d), lambda i: (i, 0)),
                 pl.BlockSpec((bb, d), lambda i: (i, 0)),
                 pl.BlockSpec((bb, 1), lambda i: (i, 0)),
                 pl.BlockSpec((1, 1), lambda i: (0, 0))]

    def body(*refs):
        (dyn, st, cd, ta, bt, wc0, b0, wr0, wc1, b1, wr1, wc2, b2) = refs[:13]
        gs_refs = refs[13:24]
        gc_refs = refs[24:34]
        outs = refs[34:]
        _prefix_body(dyn, st, cd, ta, bt, wc0, b0, wr0, wc1, b1, wr1,
                     wc2, b2, gs_refs, gc_refs, *outs)

    return pl.pallas_call(
        body, grid=grid, in_specs=in_specs, out_specs=out_specs,
        out_shape=out_shape)(*operands)


def _topk_body(qn_ref, db_ref, vals_out, idx_out, *, nb, pad):
    j = pl.program_id(0)

    db = db_ref[...]
    nrm = jnp.sqrt(jnp.sum(db * db, axis=-1, keepdims=True))
    dbn = db / (nrm + 1e-8)
    s = lax.dot_general(qn_ref[...], dbn, (((1,), (1,)), ((), ())),
                        precision=_PREC, preferred_element_type=jnp.float32)

    bq = s.shape[0]
    colf = lax.broadcasted_iota(jnp.int32, (bq, nb), 1).astype(jnp.float32)
    base = (j * nb).astype(jnp.float32)
    bvs, bis = [], []
    for t in range(TOP_K):
        m = jnp.max(s, axis=1, keepdims=True)
        am = jnp.min(jnp.where(s == m, colf, float(nb)), axis=1, keepdims=True)
        bvs.append(m)
        bis.append(am + base)
        if t < TOP_K - 1:
            s = jnp.where(colf == am, -jnp.inf, s)

    zpad = [jnp.full((bq, 1), -jnp.inf, jnp.float32)] * pad
    ipad = [jnp.zeros((bq, 1), jnp.float32)] * pad
    vals_out[...] = jnp.concatenate(bvs + zpad, axis=1)[None]
    idx_out[...] = jnp.concatenate(bis + ipad, axis=1)[None]


def _merge_body(cv_ref, ci_ref, vals_out, idx_out):
    cv = cv_ref[...]
    ci = ci_ref[...]
    big = jnp.float32(1e9)
    nvs, nis = [], []
    for t in range(TOP_K):
        m = jnp.max(cv, axis=1, keepdims=True)
        am = jnp.min(jnp.where(cv == m, ci, big), axis=1, keepdims=True)
        nvs.append(m)
        nis.append(am)
        if t < TOP_K - 1:
            cv = jnp.where(ci == am, -jnp.inf, cv)
    vals_out[...] = jnp.concatenate(nvs, axis=1)
    idx_out[...] = jnp.concatenate(nis, axis=1).astype(jnp.int32)


def _run_topk(qn, database):
    b, d = qn.shape
    db_rows = database.shape[0]
    nb = 2000 if db_rows % 2000 == 0 else db_rows
    nsteps = db_rows // nb
    kpad = 8 - TOP_K

    bvals, bidx = pl.pallas_call(
        functools.partial(_topk_body, nb=nb, pad=kpad),
        grid=(nsteps,),
        in_specs=[pl.BlockSpec((b, d), lambda j: (0, 0)),
                  pl.BlockSpec((nb, d), lambda j: (j, 0))],
        out_specs=[pl.BlockSpec((1, b, 8), lambda j: (j, 0, 0)),
                   pl.BlockSpec((1, b, 8), lambda j: (j, 0, 0))],
        out_shape=[jax.ShapeDtypeStruct((nsteps, b, 8), jnp.float32),
                   jax.ShapeDtypeStruct((nsteps, b, 8), jnp.float32)],
    )(qn, database)

    cand_v = jnp.transpose(bvals, (1, 0, 2)).reshape(b, nsteps * 8)
    cand_i = jnp.transpose(bidx, (1, 0, 2)).reshape(b, nsteps * 8)

    return pl.pallas_call(
        _merge_body,
        in_specs=[pl.BlockSpec(cand_v.shape, lambda: (0, 0)),
                  pl.BlockSpec(cand_i.shape, lambda: (0, 0))],
        out_specs=[pl.BlockSpec((b, TOP_K), lambda: (0, 0)),
                   pl.BlockSpec((b, TOP_K), lambda: (0, 0))],
        out_shape=[jax.ShapeDtypeStruct((b, TOP_K), jnp.float32),
                   jax.ShapeDtypeStruct((b, TOP_K), jnp.int32)],
    )(cand_v, cand_i)


def _run_gather_sc(database, flat_idx):
    n = flat_idx.shape[0]
    d = database.shape[1]
    info = plsc.get_sparse_core_info()
    nw = info.num_cores * info.num_subcores
    b_per_w = n // nw
    nchunks = 1
    while b_per_w // nchunks > 128 or (b_per_w // nchunks) % 8 != 0:
        nchunks += 1
    chunk = b_per_w // nchunks
    num_cores = info.num_cores

    @functools.partial(
        pl.kernel,
        mesh=plsc.VectorSubcoreMesh(core_axis_name="c", subcore_axis_name="s"),
        out_type=jax.ShapeDtypeStruct((n, d), jnp.float32),
        scratch_types=[pltpu.VMEM((b_per_w,), jnp.int32),
                       pltpu.VMEM((b_per_w, d), jnp.float32),
                       pltpu.SemaphoreType.DMA],
    )
    def gather(table_hbm, idx_hbm, out_hbm, idx_v, rows_v, sem):
        wid = lax.axis_index("s") * num_cores + lax.axis_index("c")
        base = wid * b_per_w
        pltpu.sync_copy(idx_hbm.at[pl.ds(base, b_per_w)], idx_v)
        for c in range(nchunks):
            pltpu.async_copy(table_hbm.at[idx_v.at[pl.ds(c * chunk, chunk)]],
                             rows_v.at[pl.ds(c * chunk, chunk)], sem).wait()
        pltpu.sync_copy(rows_v, out_hbm.at[pl.ds(base, b_per_w)])

    return gather(database, flat_idx)


def _tail_body(hcur_ref, hglob_ref, vals_ref, rows_ref, yprop_ref,
               fw1, fb1, fw2, fb2, fwo, fbo, dwp, dbp,
               hw1, hb1, hw2, hb2, hw3, hb3,
               pred_out, hf_out):
    v = vals_ref[...]
    w = jax.nn.softmax(v, axis=-1)
    rows = rows_ref[...]
    h_ret = jnp.zeros((rows.shape[0], rows.shape[2]), jnp.float32)
    for i in range(TOP_K):
        h_ret = h_ret + w[:, i:i + 1] * rows[:, i, :]

    h_cur = hcur_ref[...]
    h_glob = hglob_ref[...]
    h_f = ALPHA * h_ret + (1.0 - ALPHA) * h_cur
    hf_out[...] = h_f

    ssum = h_cur + h_f + h_glob
    pooled = ssum / 3.0
    a = jax.nn.relu(_mm(pooled, fw1[...]) + fb1[...])
    a = jax.nn.sigmoid(_mm(a, fw2[...]) + fb2[...])
    h_fused = _mm(ssum * a, fwo[...]) + fbo[...]

    h_prop = jax.nn.relu(_mm(yprop_ref[...], dwp[...]) + dbp[...])
    h_final = jnp.concatenate([h_fused, h_prop], axis=-1)
    x = jax.nn.relu(_mm(h_final, hw1[...]) + hb1[...])
    x = jax.nn.relu(_mm(x, hw2[...]) + hb2[...])
    pred_out[...] = _mm(x, hw3[...]) + hb3[...]


def _run_tail(h_cur, h_glob, top_vals, rows, y_prop, params):
    b, d = h_cur.shape
    fp, dp, hp = params['fusion'], params['delay'], params['head']
    ops = [h_cur, h_glob, top_vals, rows, y_prop,
           fp['W1'], fp['b1'].reshape(1, -1), fp['W2'], fp['b2'].reshape(1, -1),
           fp['Wo'], fp['bo'].reshape(1, -1),
           dp['Wp'], dp['bp'].reshape(1, -1),
           hp['W1'], hp['b1'].reshape(1, -1), hp['W2'], hp['b2'].reshape(1, -1),
           hp['W3'], hp['b3'].reshape(1, -1)]
    in_specs = [pl.BlockSpec(a.shape, lambda _n=a.ndim: (0,) * _n) for a in ops]
    return pl.pallas_call(
        _tail_body,
        in_specs=in_specs,
        out_specs=[pl.BlockSpec((b, 1), lambda: (0, 0)),
                   pl.BlockSpec((b, d), lambda: (0, 0))],
        out_shape=[jax.ShapeDtypeStruct((b, 1), jnp.float32),
                   jax.ShapeDtypeStruct((b, d), jnp.float32)],
    )(*ops)


def kernel(dynamic, static, chain_delays, turnaround_times, params, database):
    h_cur, h_glob, qn, y_prop, beta_pos = _run_prefix(
        dynamic, static, chain_delays, turnaround_times, params)
    top_vals, top_idx = _run_topk(qn, database)
    rows = _run_gather_sc(database, top_idx.reshape(-1))
    rows = rows.reshape(top_idx.shape[0], TOP_K, database.shape[1])
    pred, h_f = _run_tail(h_cur, h_glob, top_vals, rows, y_prop, params)
    return (pred[:, 0], h_cur, h_glob, h_f, y_prop[:, 0],
            beta_pos.reshape(()))
